# Initial kernel scaffold; baseline (speedup 1.0000x reference)
import functools

import jax
import jax.numpy as jnp
from jax import lax
from jax.experimental import pallas as pl
from jax.experimental.pallas import tpu as pltpu
from jax.experimental.pallas import tpu_sc as plsc

N1, N2, N3 = 50000, 10000, 2048
E1, E2 = 320000, 65536
D_IN, D_H, N_CLS = 128, 128, 64

NC, NS = 2, 16
NW = NC * NS
CHUNK = 128

C1 = 2528
E1P = C1 * CHUNK
PAD1 = E1P - E1
C2 = E2 // CHUNK

N1H = 50048
N2H = 10112
N3H = 2176
ACC1 = 10240

_mesh = plsc.VectorSubcoreMesh(core_axis_name="c", subcore_axis_name="s")


def _hist_body(s1_ref, s2_ref, d1_ref, d2_ref, zeros_ref, ones_ref,
               o_s1, o_s2, o_d1, o_d2,
               h_s1, h_s2, h_d1, h_d2, idx_v, ones_v):
    cid = lax.axis_index("c")
    sid = lax.axis_index("s")
    pltpu.sync_copy(ones_ref, ones_v)

    @pl.when(cid == 0)
    def _():
        pltpu.sync_copy(zeros_ref.at[pl.ds(0, N1H // NS)],
                        h_s1.at[pl.ds(sid * (N1H // NS), N1H // NS)])
        pltpu.sync_copy(zeros_ref.at[pl.ds(0, N2H // NS)],
                        h_s2.at[pl.ds(sid * (N2H // NS), N2H // NS)])

    @pl.when(cid == 1)
    def _():
        pltpu.sync_copy(zeros_ref.at[pl.ds(0, N2H // NS)],
                        h_d1.at[pl.ds(sid * (N2H // NS), N2H // NS)])
        pltpu.sync_copy(zeros_ref.at[pl.ds(0, N3H // NS)],
                        h_d2.at[pl.ds(sid * (N3H // NS), N3H // NS)])

    plsc.subcore_barrier()

    def _count(edge2d, hist, nchunks):
        base = sid * nchunks

        def body(j, carry):
            pltpu.sync_copy(edge2d.at[base + j], idx_v)
            pltpu.sync_copy(ones_v, hist.at[idx_v], add=True)
            return carry

        lax.fori_loop(0, nchunks, body, 0)

    @pl.when(cid == 0)
    def _():
        _count(s1_ref, h_s1, C1 // NS)
        _count(s2_ref, h_s2, C2 // NS)

    @pl.when(cid == 1)
    def _():
        _count(d1_ref, h_d1, C1 // NS)
        _count(d2_ref, h_d2, C2 // NS)

    plsc.subcore_barrier()

    @pl.when(cid == 0)
    def _():
        pltpu.sync_copy(h_s1.at[pl.ds(sid * (N1H // NS), N1H // NS)],
                        o_s1.at[pl.ds(sid * (N1H // NS), N1H // NS)])
        pltpu.sync_copy(h_s2.at[pl.ds(sid * (N2H // NS), N2H // NS)],
                        o_s2.at[pl.ds(sid * (N2H // NS), N2H // NS)])

    @pl.when(cid == 1)
    def _():
        pltpu.sync_copy(h_d1.at[pl.ds(sid * (N2H // NS), N2H // NS)],
                        o_d1.at[pl.ds(sid * (N2H // NS), N2H // NS)])
        pltpu.sync_copy(h_d2.at[pl.ds(sid * (N3H // NS), N3H // NS)],
                        o_d2.at[pl.ds(sid * (N3H // NS), N3H // NS)])


_hist_call = pl.kernel(
    _hist_body,
    out_type=[jax.ShapeDtypeStruct((N1H,), jnp.float32),
              jax.ShapeDtypeStruct((N2H,), jnp.float32),
              jax.ShapeDtypeStruct((N2H,), jnp.float32),
              jax.ShapeDtypeStruct((N3H,), jnp.float32)],
    mesh=_mesh,
    scratch_types=[pltpu.VMEM_SHARED((N1H,), jnp.float32),
                   pltpu.VMEM_SHARED((N2H,), jnp.float32),
                   pltpu.VMEM_SHARED((N2H,), jnp.float32),
                   pltpu.VMEM_SHARED((N3H,), jnp.float32),
                   pltpu.VMEM((CHUNK,), jnp.int32),
                   pltpu.VMEM((CHUNK,), jnp.float32)],
)


def _agg_body(nchunks, nacc, nout, feat_ref, src_ref, dst_ref, zeros_ref,
              out_ref, acc, src_v, dst_v, rows_v, sem):
    cid = lax.axis_index("c")
    sid = lax.axis_index("s")
    wid = sid * NC + cid
    zrows = zeros_ref.shape[0]

    rows_per_tile = nacc // NS
    for r in range(rows_per_tile // zrows):
        pltpu.sync_copy(
            zeros_ref, acc.at[pl.ds(sid * rows_per_tile + r * zrows, zrows)])
    plsc.subcore_barrier()

    base = wid * nchunks

    def body(j, carry):
        pltpu.sync_copy(src_ref.at[base + j], src_v)
        pltpu.sync_copy(dst_ref.at[base + j], dst_v)
        pltpu.make_async_copy(feat_ref.at[src_v], rows_v, sem).start()
        pltpu.make_async_copy(feat_ref.at[src_v], rows_v, sem).wait()
        pltpu.sync_copy(rows_v, acc.at[dst_v], add=True)
        return carry

    lax.fori_loop(0, nchunks, body, 0)
    plsc.subcore_barrier()

    out_rows = nout // NS
    pltpu.sync_copy(acc.at[pl.ds(sid * out_rows, out_rows)],
                    out_ref.at[pl.ds(cid * nout + sid * out_rows, out_rows)])


def _make_agg(nchunks_per_worker, nacc, nout, d):
    body = functools.partial(_agg_body, nchunks_per_worker, nacc, nout)
    return pl.kernel(
        body,
        out_type=jax.ShapeDtypeStruct((NC * nout, d), jnp.float32),
        mesh=_mesh,
        scratch_types=[pltpu.VMEM_SHARED((nacc, d), jnp.float32),
                       pltpu.VMEM((CHUNK,), jnp.int32),
                       pltpu.VMEM((CHUNK,), jnp.int32),
                       pltpu.VMEM((CHUNK, d), jnp.float32),
                       pltpu.SemaphoreType.DMA],
    )


_agg1_call = _make_agg(C1 // NW, ACC1, N2, D_H)
_agg2_call = _make_agg(C2 // NW, N3, N3, N_CLS)


def _scale_body(x_ref, deg_ref, o_ref):
    norm = lax.rsqrt(jnp.maximum(deg_ref[...], 1.0))
    o_ref[...] = x_ref[...] * norm


def _mlp_body(a0_ref, a1_ref, di_ref, do_ref, w1_ref, b1_ref, w2_ref, p_ref):
    a = a0_ref[...] + a1_ref[...]
    h = jnp.dot(a, w1_ref[...], preferred_element_type=jnp.float32)
    h = h * lax.rsqrt(jnp.maximum(di_ref[...], 1.0)) + b1_ref[...]
    h = jnp.maximum(h, 0.0)
    h = h * lax.rsqrt(jnp.maximum(do_ref[...], 1.0))
    p_ref[...] = jnp.dot(h, w2_ref[...], preferred_element_type=jnp.float32)


def _final_body(e0_ref, e1_ref, deg_ref, b2_ref, o_ref):
    agg = e0_ref[...] + e1_ref[...]
    o_ref[...] = agg * lax.rsqrt(jnp.maximum(deg_ref[...], 1.0)) + b2_ref[...]


def kernel(in_feat, mfg1_src, mfg1_dst, mfg2_src, mfg2_dst, W1, b1, W2, b2):
    i32 = jnp.int32
    s1 = mfg1_src.astype(i32)
    d1 = mfg1_dst.astype(i32)
    s2 = mfg2_src.astype(i32)
    d2 = mfg2_dst.astype(i32)

    pad = jnp.arange(PAD1, dtype=i32)
    s1h = jnp.concatenate([s1, N1 + pad % 48]).reshape(C1, CHUNK)
    s1g = jnp.concatenate([s1, jnp.zeros(PAD1, i32)]).reshape(C1, CHUNK)
    d1p = jnp.concatenate([d1, N2 + pad % 112]).reshape(C1, CHUNK)
    s2r = s2.reshape(C2, CHUNK)
    d2r = d2.reshape(C2, CHUNK)

    zeros1d = jnp.zeros((N1H // NS,), jnp.float32)
    ones1d = jnp.ones((CHUNK,), jnp.float32)
    zeros2d_h = jnp.zeros((128, D_H), jnp.float32)
    zeros2d_c = jnp.zeros((128, N_CLS), jnp.float32)

    h_s1, h_s2, h_d1, h_d2 = _hist_call(s1h, s2r, d1p, d2r, zeros1d, ones1d)
    deg1o = h_s1[:N1].reshape(N1, 1)
    deg2o = h_s2[:N2].reshape(N2, 1)
    deg1i = h_d1[:N2].reshape(N2, 1)
    deg2i = h_d2[:N3].reshape(N3, 1)

    blk = 1000
    feat_scaled = pl.pallas_call(
        _scale_body,
        grid=(N1 // blk,),
        in_specs=[pl.BlockSpec((blk, D_IN), lambda i: (i, 0)),
                  pl.BlockSpec((blk, 1), lambda i: (i, 0))],
        out_specs=pl.BlockSpec((blk, D_IN), lambda i: (i, 0)),
        out_shape=jax.ShapeDtypeStruct((N1, D_IN), jnp.float32),
        compiler_params=pltpu.CompilerParams(
            dimension_semantics=("parallel",)),
    )(in_feat, deg1o)

    agg1 = _agg1_call(feat_scaled, s1g, d1p, zeros2d_h)

    p = pl.pallas_call(
        _mlp_body,
        grid=(N2 // blk,),
        in_specs=[pl.BlockSpec((blk, D_H), lambda i: (i, 0)),
                  pl.BlockSpec((blk, D_H), lambda i: (i, 0)),
                  pl.BlockSpec((blk, 1), lambda i: (i, 0)),
                  pl.BlockSpec((blk, 1), lambda i: (i, 0)),
                  pl.BlockSpec((D_H, D_H), lambda i: (0, 0)),
                  pl.BlockSpec((1, D_H), lambda i: (0, 0)),
                  pl.BlockSpec((D_H, N_CLS), lambda i: (0, 0))],
        out_specs=pl.BlockSpec((blk, N_CLS), lambda i: (i, 0)),
        out_shape=jax.ShapeDtypeStruct((N2, N_CLS), jnp.float32),
        compiler_params=pltpu.CompilerParams(
            dimension_semantics=("parallel",)),
    )(agg1[:N2], agg1[N2:], deg1i, deg2o, W1, b1.reshape(1, D_H), W2)

    agg2 = _agg2_call(p, s2r, d2r, zeros2d_c)

    out = pl.pallas_call(
        _final_body,
        in_specs=[pl.BlockSpec((N3, N_CLS), lambda: (0, 0)),
                  pl.BlockSpec((N3, N_CLS), lambda: (0, 0)),
                  pl.BlockSpec((N3, 1), lambda: (0, 0)),
                  pl.BlockSpec((1, N_CLS), lambda: (0, 0))],
        out_specs=pl.BlockSpec((N3, N_CLS), lambda: (0, 0)),
        out_shape=jax.ShapeDtypeStruct((N3, N_CLS), jnp.float32),
    )(agg2[:N3], agg2[N3:], deg2i, b2.reshape(1, N_CLS))

    return out


# speedup vs baseline: 4.5214x; 4.5214x over previous
import functools

import jax
import jax.numpy as jnp
from jax import lax
from jax.experimental import pallas as pl
from jax.experimental.pallas import tpu as pltpu
from jax.experimental.pallas import tpu_sc as plsc

N1, N2, N3 = 50000, 10000, 2048
E1, E2 = 320000, 65536
D_IN, D_H, N_CLS = 128, 128, 64

NC, NS = 2, 16
NW = NC * NS
L = 16
CHUNK = 128

C1 = 2528
E1P = C1 * CHUNK
PAD1 = E1P - E1
C2 = E2 // CHUNK

N1H = 51200
N2H = 10240
N3H = 2048
ACC1 = 10240

_mesh = plsc.VectorSubcoreMesh(core_axis_name="c", subcore_axis_name="s")


def _fill(ref, value):
    v = jnp.full((L,), value, jnp.float32)
    if len(ref.shape) == 1:
        def body(i, c):
            ref[pl.ds(i * L, L)] = v
            return c
        lax.fori_loop(0, ref.shape[0] // L, body, 0)
    else:
        def body(r, c):
            for k in range(ref.shape[1] // L):
                ref[r, pl.ds(k * L, L)] = v
            return c
        lax.fori_loop(0, ref.shape[0], body, 0)


def _hist_body(s1_ref, s2_ref, d1_ref, d2_ref,
               o_s1, o_s2, o_d1, o_d2,
               h_s1, h_s2, h_d1, h_d2, idx_v, ones_v, buf_v):
    cid = lax.axis_index("c")
    sid = lax.axis_index("s")
    _fill(ones_v, 1.0)
    _fill(buf_v, 0.0)

    def _zero(hist):
        n = hist.shape[0] // NS
        for r in range(n // CHUNK):
            pltpu.sync_copy(buf_v.at[pl.ds(0, CHUNK)],
                            hist.at[pl.ds(sid * n + r * CHUNK, CHUNK)])

    def _count(edge2d, hist, nchunks):
        base = sid * nchunks

        def body(j, carry):
            pltpu.sync_copy(edge2d.at[base + j], idx_v)
            pltpu.sync_copy(ones_v, hist.at[idx_v], add=True)
            return carry

        lax.fori_loop(0, nchunks, body, 0)

    def _drain(hist, out):
        n = hist.shape[0] // NS
        nb = min(buf_v.shape[0], n)
        for r in range(n // nb):
            pltpu.sync_copy(hist.at[pl.ds(sid * n + r * nb, nb)],
                            buf_v.at[pl.ds(0, nb)])
            pltpu.sync_copy(buf_v.at[pl.ds(0, nb)],
                            out.at[pl.ds(sid * n + r * nb, nb)])

    @pl.when(cid == 0)
    def _():
        _zero(h_s1)
        _zero(h_s2)

    @pl.when(cid == 1)
    def _():
        _zero(h_d1)
        _zero(h_d2)

    plsc.subcore_barrier()

    @pl.when(cid == 0)
    def _():
        _count(s1_ref, h_s1, C1 // NS)
        _count(s2_ref, h_s2, C2 // NS)

    @pl.when(cid == 1)
    def _():
        _count(d1_ref, h_d1, C1 // NS)
        _count(d2_ref, h_d2, C2 // NS)

    plsc.subcore_barrier()

    @pl.when(cid == 0)
    def _():
        _drain(h_s1, o_s1)
        _drain(h_s2, o_s2)

    @pl.when(cid == 1)
    def _():
        _drain(h_d1, o_d1)
        _drain(h_d2, o_d2)


_hist_call = pl.kernel(
    _hist_body,
    out_type=[jax.ShapeDtypeStruct((N1H,), jnp.float32),
              jax.ShapeDtypeStruct((N2H,), jnp.float32),
              jax.ShapeDtypeStruct((N2H,), jnp.float32),
              jax.ShapeDtypeStruct((N3H,), jnp.float32)],
    mesh=_mesh,
    scratch_types=[pltpu.VMEM_SHARED((N1H,), jnp.float32),
                   pltpu.VMEM_SHARED((N2H,), jnp.float32),
                   pltpu.VMEM_SHARED((N2H,), jnp.float32),
                   pltpu.VMEM_SHARED((N3H,), jnp.float32),
                   pltpu.VMEM((CHUNK,), jnp.int32),
                   pltpu.VMEM((CHUNK,), jnp.float32),
                   pltpu.VMEM((640,), jnp.float32)],
)


def _agg_body(nchunks, nacc, nout, feat_ref, src_ref, dst_ref,
              out_ref, acc, src_v, dst_v, rows_v, sem):
    cid = lax.axis_index("c")
    sid = lax.axis_index("s")
    wid = sid * NC + cid

    _fill(rows_v, 0.0)
    rows_per_tile = nacc // NS
    for r in range(rows_per_tile // CHUNK):
        pltpu.sync_copy(
            rows_v, acc.at[pl.ds(sid * rows_per_tile + r * CHUNK, CHUNK)])
    plsc.subcore_barrier()

    base = wid * nchunks

    def body(j, carry):
        pltpu.sync_copy(src_ref.at[base + j], src_v)
        pltpu.sync_copy(dst_ref.at[base + j], dst_v)
        pltpu.make_async_copy(feat_ref.at[src_v], rows_v, sem).start()
        pltpu.make_async_copy(feat_ref.at[src_v], rows_v, sem).wait()
        pltpu.sync_copy(rows_v, acc.at[dst_v], add=True)
        return carry

    lax.fori_loop(0, nchunks, body, 0)
    plsc.subcore_barrier()

    out_rows = nacc // NS
    for r in range(out_rows // CHUNK):
        pltpu.sync_copy(acc.at[pl.ds(sid * out_rows + r * CHUNK, CHUNK)],
                        rows_v)
        pltpu.sync_copy(
            rows_v,
            out_ref.at[pl.ds(cid * nacc + sid * out_rows + r * CHUNK, CHUNK)])


def _make_agg(nchunks_per_worker, nacc, nout, d):
    body = functools.partial(_agg_body, nchunks_per_worker, nacc, nout)
    return pl.kernel(
        body,
        out_type=jax.ShapeDtypeStruct((NC * nacc, d), jnp.float32),
        mesh=_mesh,
        scratch_types=[pltpu.VMEM_SHARED((nacc, d), jnp.float32),
                       pltpu.VMEM((CHUNK,), jnp.int32),
                       pltpu.VMEM((CHUNK,), jnp.int32),
                       pltpu.VMEM((CHUNK, d), jnp.float32),
                       pltpu.SemaphoreType.DMA],
    )


_agg1_call = _make_agg(C1 // NW, ACC1, N2, D_H)
_agg2_call = _make_agg(C2 // NW, N3, N3, D_H)


def _scale_body(x_ref, deg_ref, o_ref):
    norm = lax.rsqrt(jnp.maximum(deg_ref[...], 1.0))
    o_ref[...] = x_ref[...] * norm


def _mlp_body(a0_ref, a1_ref, di_ref, do_ref, w1_ref, b1_ref, w2_ref, p_ref):
    a = a0_ref[...] + a1_ref[...]
    h = jnp.dot(a, w1_ref[...], preferred_element_type=jnp.float32)
    h = h * lax.rsqrt(jnp.maximum(di_ref[...], 1.0)) + b1_ref[...]
    h = jnp.maximum(h, 0.0)
    h = h * lax.rsqrt(jnp.maximum(do_ref[...], 1.0))
    p_ref[...] = jnp.dot(h, w2_ref[...], preferred_element_type=jnp.float32)


def _final_body(e0_ref, e1_ref, deg_ref, b2_ref, o_ref):
    agg = e0_ref[:, :N_CLS] + e1_ref[:, :N_CLS]
    o_ref[...] = agg * lax.rsqrt(jnp.maximum(deg_ref[...], 1.0)) + b2_ref[...]


def kernel(in_feat, mfg1_src, mfg1_dst, mfg2_src, mfg2_dst, W1, b1, W2, b2):
    i32 = jnp.int32
    s1 = mfg1_src.astype(i32)
    d1 = mfg1_dst.astype(i32)
    s2 = mfg2_src.astype(i32)
    d2 = mfg2_dst.astype(i32)

    pad = jnp.arange(PAD1, dtype=i32)
    s1h = jnp.concatenate([s1, N1 + pad % 1024]).reshape(C1, CHUNK)
    s1g = jnp.concatenate([s1, jnp.zeros(PAD1, i32)]).reshape(C1, CHUNK)
    d1p = jnp.concatenate([d1, N2 + pad % 224]).reshape(C1, CHUNK)
    s2r = s2.reshape(C2, CHUNK)
    d2r = d2.reshape(C2, CHUNK)

    h_s1, h_s2, h_d1, h_d2 = _hist_call(s1h, s2r, d1p, d2r)
    deg1o = h_s1[:N1].reshape(N1, 1)
    deg2o = h_s2[:N2].reshape(N2, 1)
    deg1i = h_d1[:N2].reshape(N2, 1)
    deg2i = h_d2[:N3].reshape(N3, 1)

    blk = 1000
    feat_scaled = pl.pallas_call(
        _scale_body,
        grid=(N1 // blk,),
        in_specs=[pl.BlockSpec((blk, D_IN), lambda i: (i, 0)),
                  pl.BlockSpec((blk, 1), lambda i: (i, 0))],
        out_specs=pl.BlockSpec((blk, D_IN), lambda i: (i, 0)),
        out_shape=jax.ShapeDtypeStruct((N1, D_IN), jnp.float32),
        compiler_params=pltpu.CompilerParams(
            dimension_semantics=("parallel",)),
    )(in_feat, deg1o)

    agg1 = _agg1_call(feat_scaled, s1g, d1p)

    p = pl.pallas_call(
        _mlp_body,
        grid=(N2 // blk,),
        in_specs=[pl.BlockSpec((blk, D_H), lambda i: (i, 0)),
                  pl.BlockSpec((blk, D_H), lambda i: (i, 0)),
                  pl.BlockSpec((blk, 1), lambda i: (i, 0)),
                  pl.BlockSpec((blk, 1), lambda i: (i, 0)),
                  pl.BlockSpec((D_H, D_H), lambda i: (0, 0)),
                  pl.BlockSpec((1, D_H), lambda i: (0, 0)),
                  pl.BlockSpec((D_H, D_H), lambda i: (0, 0))],
        out_specs=pl.BlockSpec((blk, D_H), lambda i: (i, 0)),
        out_shape=jax.ShapeDtypeStruct((N2, D_H), jnp.float32),
        compiler_params=pltpu.CompilerParams(
            dimension_semantics=("parallel",)),
    )(agg1[:N2], agg1[ACC1:ACC1 + N2], deg1i, deg2o, W1,
      b1.reshape(1, D_H), jnp.pad(W2, ((0, 0), (0, D_H - N_CLS))))

    agg2 = _agg2_call(p, s2r, d2r)

    out = pl.pallas_call(
        _final_body,
        in_specs=[pl.BlockSpec((N3, D_H), lambda: (0, 0)),
                  pl.BlockSpec((N3, D_H), lambda: (0, 0)),
                  pl.BlockSpec((N3, 1), lambda: (0, 0)),
                  pl.BlockSpec((1, N_CLS), lambda: (0, 0))],
        out_specs=pl.BlockSpec((N3, N_CLS), lambda: (0, 0)),
        out_shape=jax.ShapeDtypeStruct((N3, N_CLS), jnp.float32),
    )(agg2[:N3], agg2[N3:2 * N3], deg2i, b2.reshape(1, N_CLS))

    return out


# speedup vs baseline: 10.6862x; 2.3634x over previous
import functools

import jax
import jax.numpy as jnp
from jax import lax
from jax.experimental import pallas as pl
from jax.experimental.pallas import tpu as pltpu
from jax.experimental.pallas import tpu_sc as plsc

N1, N2, N3 = 50000, 10000, 2048
E1, E2 = 320000, 65536
D_IN, D_H, N_CLS = 128, 128, 64

NC, NS = 2, 16
NW = NC * NS
L = 16
CHUNK = 128

C1 = 2560
E1P = C1 * CHUNK
PAD1 = E1P - E1
C2 = E2 // CHUNK
KBUF = 2
WIN = 32

N1H = 51200
N2H = 10240
N3H = 2048
ACC1 = 10240

_mesh = plsc.VectorSubcoreMesh(core_axis_name="c", subcore_axis_name="s")


def _fill(ref, value):
    v = jnp.full((L,), value, jnp.float32)
    if len(ref.shape) == 1:
        def body(i, c):
            ref[pl.ds(i * L, L)] = v
            return c
        lax.fori_loop(0, ref.shape[0] // L, body, 0)
    else:
        def body(r, c):
            for k in range(ref.shape[1] // L):
                ref[r, pl.ds(k * L, L)] = v
            return c
        lax.fori_loop(0, ref.shape[0], body, 0)


def _hist_body(s1_ref, s2_ref, d1_ref, d2_ref,
               o_s1, o_s2, o_d1, o_d2,
               h_s1, h_s2, h_d1, h_d2, idx_v, ones_v, buf_v, sem):
    cid = lax.axis_index("c")
    sid = lax.axis_index("s")
    _fill(ones_v, 1.0)
    _fill(buf_v, 0.0)

    def _zero(hist):
        n = hist.shape[0] // NS
        for r in range(n // CHUNK):
            pltpu.sync_copy(buf_v.at[pl.ds(0, CHUNK)],
                            hist.at[pl.ds(sid * n + r * CHUNK, CHUNK)])

    def _count(edge2d, hist, nchunks):
        nph = idx_v.shape[0]

        def fire(j, carry):
            pltpu.sync_copy(ones_v, hist.at[idx_v.at[j]], add=True)
            return carry

        for ph in range(nchunks // nph):
            pltpu.sync_copy(edge2d.at[pl.ds(sid * nchunks + ph * nph, nph)],
                            idx_v)
            lax.fori_loop(0, nph, fire, 0)

    def _drain(hist, out):
        n = hist.shape[0] // NS
        nb = min(buf_v.shape[0], n)
        for r in range(n // nb):
            pltpu.sync_copy(hist.at[pl.ds(sid * n + r * nb, nb)],
                            buf_v.at[pl.ds(0, nb)])
            pltpu.sync_copy(buf_v.at[pl.ds(0, nb)],
                            out.at[pl.ds(sid * n + r * nb, nb)])

    @pl.when(cid == 0)
    def _():
        _zero(h_s1)
        _zero(h_s2)

    @pl.when(cid == 1)
    def _():
        _zero(h_d1)
        _zero(h_d2)

    plsc.subcore_barrier()

    @pl.when(cid == 0)
    def _():
        _count(s1_ref, h_s1, C1 // NS)
        _count(s2_ref, h_s2, C2 // NS)

    @pl.when(cid == 1)
    def _():
        _count(d1_ref, h_d1, C1 // NS)
        _count(d2_ref, h_d2, C2 // NS)

    plsc.subcore_barrier()

    @pl.when(cid == 0)
    def _():
        _drain(h_s1, o_s1)
        _drain(h_s2, o_s2)

    @pl.when(cid == 1)
    def _():
        _drain(h_d1, o_d1)
        _drain(h_d2, o_d2)


_hist_call = pl.kernel(
    _hist_body,
    out_type=[jax.ShapeDtypeStruct((N1H,), jnp.float32),
              jax.ShapeDtypeStruct((N2H,), jnp.float32),
              jax.ShapeDtypeStruct((N2H,), jnp.float32),
              jax.ShapeDtypeStruct((N3H,), jnp.float32)],
    mesh=_mesh,
    scratch_types=[pltpu.VMEM_SHARED((N1H,), jnp.float32),
                   pltpu.VMEM_SHARED((N2H,), jnp.float32),
                   pltpu.VMEM_SHARED((N2H,), jnp.float32),
                   pltpu.VMEM_SHARED((N3H,), jnp.float32),
                   pltpu.VMEM((32, CHUNK), jnp.int32),
                   pltpu.VMEM((CHUNK,), jnp.float32),
                   pltpu.VMEM((640,), jnp.float32),
                   pltpu.SemaphoreType.DMA],
)


def _agg_body(nchunks, nphase, nacc, feat_ref, src_ref, dst_ref,
              out_ref, acc, src_v, dst_v, r0, r1, s0, s1):
    cid = lax.axis_index("c")
    sid = lax.axis_index("s")
    wid = sid * NC + cid
    rows = [r0, r1]
    sems = [s0, s1]
    nph = nchunks // nphase

    _fill(r0, 0.0)
    rows_per_tile = nacc // NS
    for r in range(rows_per_tile // CHUNK):
        pltpu.sync_copy(
            r0, acc.at[pl.ds(sid * rows_per_tile + r * CHUNK, CHUNK)])
    plsc.subcore_barrier()

    def _gather(j, k):
        return pltpu.make_async_copy(feat_ref.at[src_v.at[j]],
                                     rows[k], sems[k])

    for ph in range(nphase):
        base = wid * nchunks + ph * nph
        pltpu.sync_copy(src_ref.at[pl.ds(base, nph)], src_v)
        pltpu.sync_copy(dst_ref.at[pl.ds(base, nph)], dst_v)

        for k in range(KBUF):
            _gather(k, k).start()

        nblk = nph // KBUF

        def body(blk, carry):
            for k in range(KBUF):
                j = blk * KBUF + k
                _gather(j, k).wait()
                pltpu.sync_copy(rows[k], acc.at[dst_v.at[j]], add=True)

                @pl.when(blk < nblk - 1)
                def _():
                    _gather(j + KBUF, k).start()
            return carry

        lax.fori_loop(0, nblk, body, 0)

    plsc.subcore_barrier()

    out_rows = nacc // NS
    for r in range(out_rows // CHUNK):
        pltpu.sync_copy(acc.at[pl.ds(sid * out_rows + r * CHUNK, CHUNK)], r0)
        pltpu.sync_copy(
            r0,
            out_ref.at[pl.ds(cid * nacc + sid * out_rows + r * CHUNK, CHUNK)])


def _make_agg(nchunks_per_worker, nphase, nacc, d):
    body = functools.partial(_agg_body, nchunks_per_worker, nphase, nacc)
    nph = nchunks_per_worker // nphase
    return pl.kernel(
        body,
        out_type=jax.ShapeDtypeStruct((NC * nacc, d), jnp.float32),
        mesh=_mesh,
        scratch_types=[pltpu.VMEM_SHARED((nacc, d), jnp.float32),
                       pltpu.VMEM((nph, CHUNK), jnp.int32),
                       pltpu.VMEM((nph, CHUNK), jnp.int32)]
                      + [pltpu.VMEM((CHUNK, d), jnp.float32)] * KBUF
                      + [pltpu.SemaphoreType.DMA] * KBUF,
    )


_agg1_call = _make_agg(C1 // NW, 2, ACC1, D_H)
_agg2_call = _make_agg(C2 // NW, 2, N3, D_H)


def _scale_body(x_ref, deg_ref, o_ref):
    norm = lax.rsqrt(jnp.maximum(deg_ref[...], 1.0))
    o_ref[...] = x_ref[...] * norm


def _mlp_body(a0_ref, a1_ref, di_ref, do_ref, w1_ref, b1_ref, w2_ref, p_ref):
    a = a0_ref[...] + a1_ref[...]
    h = jnp.dot(a, w1_ref[...], preferred_element_type=jnp.float32)
    h = h * lax.rsqrt(jnp.maximum(di_ref[...], 1.0)) + b1_ref[...]
    h = jnp.maximum(h, 0.0)
    h = h * lax.rsqrt(jnp.maximum(do_ref[...], 1.0))
    p_ref[...] = jnp.dot(h, w2_ref[...], preferred_element_type=jnp.float32)


def _final_body(e0_ref, e1_ref, deg_ref, b2_ref, o_ref):
    agg = e0_ref[:, :N_CLS] + e1_ref[:, :N_CLS]
    o_ref[...] = agg * lax.rsqrt(jnp.maximum(deg_ref[...], 1.0)) + b2_ref[...]


def kernel(in_feat, mfg1_src, mfg1_dst, mfg2_src, mfg2_dst, W1, b1, W2, b2):
    i32 = jnp.int32
    s1 = mfg1_src.astype(i32)
    d1 = mfg1_dst.astype(i32)
    s2 = mfg2_src.astype(i32)
    d2 = mfg2_dst.astype(i32)

    pad = jnp.arange(PAD1, dtype=i32)
    s1h = jnp.concatenate([s1, N1 + pad % 1024]).reshape(C1, CHUNK)
    s1g = jnp.concatenate([s1, pad % N1]).reshape(C1, CHUNK)
    d1p = jnp.concatenate([d1, N2 + pad % 224]).reshape(C1, CHUNK)
    s2r = s2.reshape(C2, CHUNK)
    d2r = d2.reshape(C2, CHUNK)

    h_s1, h_s2, h_d1, h_d2 = _hist_call(s1h, s2r, d1p, d2r)
    deg1o = h_s1[:N1].reshape(N1, 1)
    deg2o = h_s2[:N2].reshape(N2, 1)
    deg1i = h_d1[:N2].reshape(N2, 1)
    deg2i = h_d2[:N3].reshape(N3, 1)

    blk = 1000
    feat_scaled = pl.pallas_call(
        _scale_body,
        grid=(N1 // blk,),
        in_specs=[pl.BlockSpec((blk, D_IN), lambda i: (i, 0)),
                  pl.BlockSpec((blk, 1), lambda i: (i, 0))],
        out_specs=pl.BlockSpec((blk, D_IN), lambda i: (i, 0)),
        out_shape=jax.ShapeDtypeStruct((N1, D_IN), jnp.float32),
        compiler_params=pltpu.CompilerParams(
            dimension_semantics=("parallel",)),
    )(in_feat, deg1o)

    agg1 = _agg1_call(feat_scaled, s1g, d1p)

    p = pl.pallas_call(
        _mlp_body,
        grid=(N2 // blk,),
        in_specs=[pl.BlockSpec((blk, D_H), lambda i: (i, 0)),
                  pl.BlockSpec((blk, D_H), lambda i: (i, 0)),
                  pl.BlockSpec((blk, 1), lambda i: (i, 0)),
                  pl.BlockSpec((blk, 1), lambda i: (i, 0)),
                  pl.BlockSpec((D_H, D_H), lambda i: (0, 0)),
                  pl.BlockSpec((1, D_H), lambda i: (0, 0)),
                  pl.BlockSpec((D_H, D_H), lambda i: (0, 0))],
        out_specs=pl.BlockSpec((blk, D_H), lambda i: (i, 0)),
        out_shape=jax.ShapeDtypeStruct((N2, D_H), jnp.float32),
        compiler_params=pltpu.CompilerParams(
            dimension_semantics=("parallel",)),
    )(agg1[:N2], agg1[ACC1:ACC1 + N2], deg1i, deg2o, W1,
      b1.reshape(1, D_H), jnp.pad(W2, ((0, 0), (0, D_H - N_CLS))))

    agg2 = _agg2_call(p, s2r, d2r)

    out = pl.pallas_call(
        _final_body,
        in_specs=[pl.BlockSpec((N3, D_H), lambda: (0, 0)),
                  pl.BlockSpec((N3, D_H), lambda: (0, 0)),
                  pl.BlockSpec((N3, 1), lambda: (0, 0)),
                  pl.BlockSpec((1, N_CLS), lambda: (0, 0))],
        out_specs=pl.BlockSpec((N3, N_CLS), lambda: (0, 0)),
        out_shape=jax.ShapeDtypeStruct((N3, N_CLS), jnp.float32),
    )(agg2[:N3], agg2[N3:2 * N3], deg2i, b2.reshape(1, N_CLS))

    return out
